# Initial kernel scaffold; baseline (speedup 1.0000x reference)
import jax
import jax.numpy as jnp
from jax.experimental import pallas as pl


def kernel(x, edge_index, edge_attr, batch, logmw, W0, b0, W1, b1, W2, b2, gamma, beta, fcW, fcb):
    raise NotImplementedError("write your pallas kernel here")


# speedup vs baseline: 3.9128x; 3.9128x over previous
import functools

import jax
import jax.numpy as jnp
from jax import lax
from jax.experimental import pallas as pl
from jax.experimental.pallas import tpu as pltpu
from jax.experimental.pallas import tpu_sc as plsc

N = 10000
E_RAW = 160000
E_PAD = 163840
BLK = 128
RPT = 624
TAIL = N - 16 * RPT
EPT = E_PAD // 16
EPW = E_PAD // 32
R = 1000

_mesh = plsc.VectorSubcoreMesh(core_axis_name="c", subcore_axis_name="s")


@functools.partial(
    pl.kernel,
    mesh=_mesh,
    out_type=jax.ShapeDtypeStruct((2 * N, 16), jnp.float32),
    scratch_types=[
        pltpu.VMEM((BLK,), jnp.int32),
        pltpu.VMEM((BLK, 16), jnp.float32),
        pltpu.VMEM_SHARED((N, 16), jnp.float32),
        pltpu.SemaphoreType.DMA,
    ],
)
def _deg_kernel(ones_hbm, dst_hbm, ew16_hbm, out_hbm, didx, vals, acc, sem):
    c = lax.axis_index("c")
    s = lax.axis_index("s")
    w = s * 2 + c
    pltpu.sync_copy(ones_hbm.at[pl.ds(s * RPT, RPT)], acc.at[pl.ds(s * RPT, RPT)])

    @pl.when(s == 15)
    def _():
        pltpu.sync_copy(ones_hbm.at[pl.ds(16 * RPT, TAIL)],
                        acc.at[pl.ds(16 * RPT, TAIL)])

    plsc.subcore_barrier()

    def blk(j, carry):
        base = w * EPW + j * BLK
        pltpu.sync_copy(dst_hbm.at[pl.ds(base, BLK)], didx)
        pltpu.sync_copy(ew16_hbm.at[pl.ds(base, BLK)], vals)
        pltpu.sync_copy(vals, acc.at[didx], add=True)
        return carry

    lax.fori_loop(0, EPW // BLK, blk, 0)
    plsc.subcore_barrier()
    pltpu.sync_copy(
        acc.at[pl.ds(s * RPT, RPT)], out_hbm.at[pl.ds(c * N + s * RPT, RPT)]
    )

    @pl.when(s == 15)
    def _():
        pltpu.sync_copy(acc.at[pl.ds(16 * RPT, TAIL)],
                        out_hbm.at[pl.ds(c * N + 16 * RPT, TAIL)])


@functools.partial(
    pl.kernel,
    mesh=_mesh,
    out_type=jax.ShapeDtypeStruct((2 * N, 128), jnp.float32),
    scratch_types=[
        pltpu.VMEM((BLK,), jnp.int32),
        pltpu.VMEM((BLK,), jnp.int32),
        pltpu.VMEM((BLK, 16), jnp.float32),
        pltpu.VMEM((BLK, 128), jnp.float32),
        pltpu.VMEM_SHARED((N, 128), jnp.float32),
        pltpu.SemaphoreType.DMA,
    ],
)
def _conv_kernel(gflat_hbm, src_hbm, dst_hbm, ew16_hbm, out_hbm,
                 sidx, didx, ewm, rows, acc, sem):
    c = lax.axis_index("c")
    s = lax.axis_index("s")
    off = c * N
    pltpu.sync_copy(
        gflat_hbm.at[pl.ds(off + s * RPT, RPT)], acc.at[pl.ds(s * RPT, RPT)]
    )

    @pl.when(s == 15)
    def _():
        pltpu.sync_copy(gflat_hbm.at[pl.ds(off + 16 * RPT, TAIL)],
                        acc.at[pl.ds(16 * RPT, TAIL)])

    plsc.subcore_barrier()

    def blk(j, carry):
        base = s * EPT + j * BLK
        pltpu.sync_copy(src_hbm.at[pl.ds(base, BLK)], sidx)
        pltpu.sync_copy(dst_hbm.at[pl.ds(base, BLK)], didx)
        pltpu.sync_copy(ew16_hbm.at[pl.ds(base, BLK)], ewm)
        for v in range(BLK // 16):
            sidx[pl.ds(v * 16, 16)] = sidx[pl.ds(v * 16, 16)] + off
        pltpu.async_copy(gflat_hbm.at[sidx], rows, sem).wait()

        def scale(jj, cc):
            wv = ewm[jj]
            for v in range(8):
                rows[jj, pl.ds(v * 16, 16)] = rows[jj, pl.ds(v * 16, 16)] * wv
            return cc

        lax.fori_loop(0, BLK, scale, 0)
        pltpu.sync_copy(rows, acc.at[didx], add=True)
        return carry

    lax.fori_loop(0, EPT // BLK, blk, 0)
    plsc.subcore_barrier()
    pltpu.sync_copy(
        acc.at[pl.ds(s * RPT, RPT)], out_hbm.at[pl.ds(off + s * RPT, RPT)]
    )

    @pl.when(s == 15)
    def _():
        pltpu.sync_copy(acc.at[pl.ds(16 * RPT, TAIL)],
                        out_hbm.at[pl.ds(off + 16 * RPT, TAIL)])


def _pre_body(x_ref, degp_ref, w_ref, g2_ref, dis_ref):
    deg = degp_ref[0, :, 0:1] + degp_ref[1, :, 0:1] - 1.0
    dis = lax.rsqrt(deg)
    h = jnp.dot(x_ref[...], w_ref[...], preferred_element_type=jnp.float32)
    g = h * dis
    g2_ref[0] = g[:, :128]
    g2_ref[1] = g[:, 128:]
    dis_ref[...] = dis


def _norm_relu(acc_ref, dis, b_ref, gamma_ref, beta_ref):
    y = jnp.concatenate([acc_ref[0], acc_ref[1]], axis=1) * dis + b_ref[...]
    mu = jnp.mean(y, axis=-1, keepdims=True)
    var = jnp.mean((y - mu) * (y - mu), axis=-1, keepdims=True)
    yn = (y - mu) * lax.rsqrt(var + 1e-5) * gamma_ref[...] + beta_ref[...]
    return jnp.maximum(yn, 0.0)


def _mid_body(acc_ref, dis_ref, b_ref, gamma_ref, beta_ref, w_ref, g2_ref):
    dis = dis_ref[...]
    yr = _norm_relu(acc_ref, dis, b_ref, gamma_ref, beta_ref)
    g = jnp.dot(yr, w_ref[...], preferred_element_type=jnp.float32) * dis
    g2_ref[0] = g[:, :128]
    g2_ref[1] = g[:, 128:]


def _post_body(acc_ref, dis_ref, b_ref, gamma_ref, beta_ref, batch_ref,
               logmw_ref, fw1_ref, fw2_ref, fb_ref, out_ref, sums_ref, cnt_ref):
    i = pl.program_id(0)
    yr = _norm_relu(acc_ref, dis_ref[...], b_ref, gamma_ref, beta_ref)
    seg = batch_ref[...]
    gid = lax.broadcasted_iota(jnp.int32, (R, 64), 1)
    mask = (gid == seg).astype(jnp.float32)
    dn = (((0,), (0,)), ((), ()))
    psum = lax.dot_general(mask, yr, dn, preferred_element_type=jnp.float32)
    pcnt = lax.dot_general(mask, jnp.ones((R, 1), jnp.float32), dn,
                           preferred_element_type=jnp.float32)

    @pl.when(i == 0)
    def _():
        sums_ref[...] = psum
        cnt_ref[...] = pcnt

    @pl.when(i != 0)
    def _():
        sums_ref[...] = sums_ref[...] + psum
        cnt_ref[...] = cnt_ref[...] + pcnt

    @pl.when(i == (N // R) - 1)
    def _():
        pooled = sums_ref[...] / jnp.maximum(cnt_ref[...], 1.0)
        hv = (jnp.dot(pooled, fw1_ref[...], preferred_element_type=jnp.float32)
              + logmw_ref[...] * fw2_ref[...] + fb_ref[...])
        out_ref[...] = jnp.tanh(hv)


_pre_tc = pl.pallas_call(
    _pre_body,
    grid=(N // R,),
    in_specs=[
        pl.BlockSpec((R, 256), lambda i: (i, 0)),
        pl.BlockSpec((2, R, 16), lambda i: (0, i, 0)),
        pl.BlockSpec((256, 256), lambda i: (0, 0)),
    ],
    out_specs=[
        pl.BlockSpec((2, R, 128), lambda i: (0, i, 0)),
        pl.BlockSpec((R, 1), lambda i: (i, 0)),
    ],
    out_shape=[
        jax.ShapeDtypeStruct((2, N, 128), jnp.float32),
        jax.ShapeDtypeStruct((N, 1), jnp.float32),
    ],
)

_mid_tc = pl.pallas_call(
    _mid_body,
    grid=(N // R,),
    in_specs=[
        pl.BlockSpec((2, R, 128), lambda i: (0, i, 0)),
        pl.BlockSpec((R, 1), lambda i: (i, 0)),
        pl.BlockSpec((1, 256), lambda i: (0, 0)),
        pl.BlockSpec((1, 256), lambda i: (0, 0)),
        pl.BlockSpec((1, 256), lambda i: (0, 0)),
        pl.BlockSpec((256, 256), lambda i: (0, 0)),
    ],
    out_specs=pl.BlockSpec((2, R, 128), lambda i: (0, i, 0)),
    out_shape=jax.ShapeDtypeStruct((2, N, 128), jnp.float32),
)

_post_tc = pl.pallas_call(
    _post_body,
    grid=(N // R,),
    in_specs=[
        pl.BlockSpec((2, R, 128), lambda i: (0, i, 0)),
        pl.BlockSpec((R, 1), lambda i: (i, 0)),
        pl.BlockSpec((1, 256), lambda i: (0, 0)),
        pl.BlockSpec((1, 256), lambda i: (0, 0)),
        pl.BlockSpec((1, 256), lambda i: (0, 0)),
        pl.BlockSpec((R, 1), lambda i: (i, 0)),
        pl.BlockSpec((64, 1), lambda i: (0, 0)),
        pl.BlockSpec((256, 128), lambda i: (0, 0)),
        pl.BlockSpec((1, 128), lambda i: (0, 0)),
        pl.BlockSpec((1, 128), lambda i: (0, 0)),
    ],
    out_specs=pl.BlockSpec((64, 128), lambda i: (0, 0)),
    out_shape=jax.ShapeDtypeStruct((64, 128), jnp.float32),
    scratch_shapes=[
        pltpu.VMEM((64, 256), jnp.float32),
        pltpu.VMEM((64, 1), jnp.float32),
    ],
)


def kernel(x, edge_index, edge_attr, batch, logmw,
           W0, b0, W1, b1, W2, b2, gamma, beta, fcW, fcb):
    src = edge_index[0].astype(jnp.int32)
    dst = edge_index[1].astype(jnp.int32)
    ew = edge_attr.astype(jnp.float32)
    pad = E_PAD - src.shape[0]
    src_p = jnp.pad(src, (0, pad))
    dst_p = jnp.pad(dst, (0, pad))
    ew_p = jnp.pad(ew, (0, pad))
    ew16 = jnp.broadcast_to(ew_p[:, None], (E_PAD, 16))
    ones16 = jnp.ones((N, 16), jnp.float32)

    degp = _deg_kernel(ones16, dst_p, ew16).reshape(2, N, 16)
    g2, dis = _pre_tc(x, degp, W0)

    acc = _conv_kernel(g2.reshape(2 * N, 128), src_p, dst_p, ew16)
    g2 = _mid_tc(acc.reshape(2, N, 128), dis, b0.reshape(1, 256),
                 gamma.reshape(1, 256), beta.reshape(1, 256), W1)
    acc = _conv_kernel(g2.reshape(2 * N, 128), src_p, dst_p, ew16)
    g2 = _mid_tc(acc.reshape(2, N, 128), dis, b1.reshape(1, 256),
                 gamma.reshape(1, 256), beta.reshape(1, 256), W2)
    acc = _conv_kernel(g2.reshape(2 * N, 128), src_p, dst_p, ew16)

    return _post_tc(acc.reshape(2, N, 128), dis, b2.reshape(1, 256),
                    gamma.reshape(1, 256), beta.reshape(1, 256),
                    batch.astype(jnp.int32).reshape(N, 1), logmw,
                    fcW[:256], fcW[256:].reshape(1, 128),
                    fcb.reshape(1, 128))


# speedup vs baseline: 4.3216x; 1.1045x over previous
import functools

import jax
import jax.numpy as jnp
from jax import lax
from jax.experimental import pallas as pl
from jax.experimental.pallas import tpu as pltpu
from jax.experimental.pallas import tpu_sc as plsc

N = 10000
E_RAW = 160000
E_PAD = 163840
BLK = 64
RPT = 624
TAIL = N - 16 * RPT
EPT = E_PAD // 16
EPW = E_PAD // 32
R = 1000

_mesh = plsc.VectorSubcoreMesh(core_axis_name="c", subcore_axis_name="s")


@functools.partial(
    pl.kernel,
    mesh=_mesh,
    out_type=jax.ShapeDtypeStruct((2 * N, 16), jnp.float32),
    scratch_types=[
        pltpu.VMEM((BLK,), jnp.int32),
        pltpu.VMEM((BLK, 16), jnp.float32),
        pltpu.VMEM_SHARED((N, 16), jnp.float32),
        pltpu.SemaphoreType.DMA,
    ],
)
def _deg_kernel(ones_hbm, dst_hbm, ew16_hbm, out_hbm, didx, vals, acc, sem):
    c = lax.axis_index("c")
    s = lax.axis_index("s")
    w = s * 2 + c
    pltpu.sync_copy(ones_hbm.at[pl.ds(s * RPT, RPT)], acc.at[pl.ds(s * RPT, RPT)])

    @pl.when(s == 15)
    def _():
        pltpu.sync_copy(ones_hbm.at[pl.ds(16 * RPT, TAIL)],
                        acc.at[pl.ds(16 * RPT, TAIL)])

    plsc.subcore_barrier()

    def blk(j, carry):
        base = w * EPW + j * BLK
        pltpu.sync_copy(dst_hbm.at[pl.ds(base, BLK)], didx)
        pltpu.sync_copy(ew16_hbm.at[pl.ds(base, BLK)], vals)
        pltpu.sync_copy(vals, acc.at[didx], add=True)
        return carry

    lax.fori_loop(0, EPW // BLK, blk, 0)
    plsc.subcore_barrier()
    pltpu.sync_copy(
        acc.at[pl.ds(s * RPT, RPT)], out_hbm.at[pl.ds(c * N + s * RPT, RPT)]
    )

    @pl.when(s == 15)
    def _():
        pltpu.sync_copy(acc.at[pl.ds(16 * RPT, TAIL)],
                        out_hbm.at[pl.ds(c * N + 16 * RPT, TAIL)])


NBUF = 2


@functools.partial(
    pl.kernel,
    mesh=_mesh,
    out_type=jax.ShapeDtypeStruct((2 * N, 128), jnp.float32),
    scratch_types=(
        [pltpu.VMEM((BLK,), jnp.int32)] * NBUF
        + [pltpu.VMEM((BLK,), jnp.int32)] * NBUF
        + [pltpu.VMEM((BLK, 16), jnp.float32)] * NBUF
        + [pltpu.VMEM((BLK, 128), jnp.float32)] * NBUF
        + [pltpu.VMEM_SHARED((N, 128), jnp.float32)]
        + [pltpu.SemaphoreType.DMA] * (3 * NBUF)
    ),
)
def _conv_kernel(gflat_hbm, src_hbm, dst_hbm, ew16_hbm, out_hbm, *scr):
    sidx = scr[0:NBUF]
    didx = scr[NBUF:2 * NBUF]
    ewm = scr[2 * NBUF:3 * NBUF]
    rows = scr[3 * NBUF:4 * NBUF]
    acc = scr[4 * NBUF]
    sem_ld = scr[4 * NBUF + 1:4 * NBUF + 1 + NBUF]
    sem_g = scr[4 * NBUF + 1 + NBUF:4 * NBUF + 1 + 2 * NBUF]
    sem_sc = scr[4 * NBUF + 1 + 2 * NBUF:]

    c = lax.axis_index("c")
    s = lax.axis_index("s")
    off = c * N
    pltpu.sync_copy(
        gflat_hbm.at[pl.ds(off + s * RPT, RPT)], acc.at[pl.ds(s * RPT, RPT)]
    )

    @pl.when(s == 15)
    def _():
        pltpu.sync_copy(gflat_hbm.at[pl.ds(off + 16 * RPT, TAIL)],
                        acc.at[pl.ds(16 * RPT, TAIL)])

    plsc.subcore_barrier()

    def superblk(j, carry):
        lh = []
        for b in range(NBUF):
            base = s * EPT + (j * NBUF + b) * BLK
            lh.append((
                pltpu.async_copy(src_hbm.at[pl.ds(base, BLK)], sidx[b], sem_ld[b]),
                pltpu.async_copy(dst_hbm.at[pl.ds(base, BLK)], didx[b], sem_ld[b]),
                pltpu.async_copy(ew16_hbm.at[pl.ds(base, BLK)], ewm[b], sem_ld[b]),
            ))
        gh = []
        for b in range(NBUF):
            for h in lh[b]:
                h.wait()
            for v in range(BLK // 16):
                sidx[b][pl.ds(v * 16, 16)] = sidx[b][pl.ds(v * 16, 16)] + off
            gh.append(pltpu.async_copy(gflat_hbm.at[sidx[b]], rows[b], sem_g[b]))
        sh = []
        for b in range(NBUF):
            gh[b].wait()
            rows_b, ewm_b = rows[b], ewm[b]

            def scale(jj, cc):
                wv = ewm_b[jj]
                for v in range(8):
                    rows_b[jj, pl.ds(v * 16, 16)] = (
                        rows_b[jj, pl.ds(v * 16, 16)] * wv)
                return cc

            lax.fori_loop(0, BLK, scale, 0, unroll=4)
            sh.append(pltpu.async_copy(rows[b], acc.at[didx[b]], sem_sc[b],
                                       add=True))
        for h in sh:
            h.wait()
        return carry

    lax.fori_loop(0, EPT // (BLK * NBUF), superblk, 0)
    plsc.subcore_barrier()
    pltpu.sync_copy(
        acc.at[pl.ds(s * RPT, RPT)], out_hbm.at[pl.ds(off + s * RPT, RPT)]
    )

    @pl.when(s == 15)
    def _():
        pltpu.sync_copy(acc.at[pl.ds(16 * RPT, TAIL)],
                        out_hbm.at[pl.ds(off + 16 * RPT, TAIL)])


def _pre_body(x_ref, degp_ref, w_ref, g2_ref, dis_ref):
    deg = degp_ref[0, :, 0:1] + degp_ref[1, :, 0:1] - 1.0
    dis = lax.rsqrt(deg)
    h = jnp.dot(x_ref[...], w_ref[...], preferred_element_type=jnp.float32)
    g = h * dis
    g2_ref[0] = g[:, :128]
    g2_ref[1] = g[:, 128:]
    dis_ref[...] = dis


def _norm_relu(acc_ref, dis, b_ref, gamma_ref, beta_ref):
    y = jnp.concatenate([acc_ref[0], acc_ref[1]], axis=1) * dis + b_ref[...]
    mu = jnp.mean(y, axis=-1, keepdims=True)
    var = jnp.mean((y - mu) * (y - mu), axis=-1, keepdims=True)
    yn = (y - mu) * lax.rsqrt(var + 1e-5) * gamma_ref[...] + beta_ref[...]
    return jnp.maximum(yn, 0.0)


def _mid_body(acc_ref, dis_ref, b_ref, gamma_ref, beta_ref, w_ref, g2_ref):
    dis = dis_ref[...]
    yr = _norm_relu(acc_ref, dis, b_ref, gamma_ref, beta_ref)
    g = jnp.dot(yr, w_ref[...], preferred_element_type=jnp.float32) * dis
    g2_ref[0] = g[:, :128]
    g2_ref[1] = g[:, 128:]


def _post_body(acc_ref, dis_ref, b_ref, gamma_ref, beta_ref, batch_ref,
               logmw_ref, fw1_ref, fw2_ref, fb_ref, out_ref, sums_ref, cnt_ref):
    i = pl.program_id(0)
    yr = _norm_relu(acc_ref, dis_ref[...], b_ref, gamma_ref, beta_ref)
    seg = batch_ref[...]
    gid = lax.broadcasted_iota(jnp.int32, (R, 64), 1)
    mask = (gid == seg).astype(jnp.float32)
    dn = (((0,), (0,)), ((), ()))
    psum = lax.dot_general(mask, yr, dn, preferred_element_type=jnp.float32)
    pcnt = lax.dot_general(mask, jnp.ones((R, 1), jnp.float32), dn,
                           preferred_element_type=jnp.float32)

    @pl.when(i == 0)
    def _():
        sums_ref[...] = psum
        cnt_ref[...] = pcnt

    @pl.when(i != 0)
    def _():
        sums_ref[...] = sums_ref[...] + psum
        cnt_ref[...] = cnt_ref[...] + pcnt

    @pl.when(i == (N // R) - 1)
    def _():
        pooled = sums_ref[...] / jnp.maximum(cnt_ref[...], 1.0)
        hv = (jnp.dot(pooled, fw1_ref[...], preferred_element_type=jnp.float32)
              + logmw_ref[...] * fw2_ref[...] + fb_ref[...])
        out_ref[...] = jnp.tanh(hv)


_pre_tc = pl.pallas_call(
    _pre_body,
    grid=(N // R,),
    in_specs=[
        pl.BlockSpec((R, 256), lambda i: (i, 0)),
        pl.BlockSpec((2, R, 16), lambda i: (0, i, 0)),
        pl.BlockSpec((256, 256), lambda i: (0, 0)),
    ],
    out_specs=[
        pl.BlockSpec((2, R, 128), lambda i: (0, i, 0)),
        pl.BlockSpec((R, 1), lambda i: (i, 0)),
    ],
    out_shape=[
        jax.ShapeDtypeStruct((2, N, 128), jnp.float32),
        jax.ShapeDtypeStruct((N, 1), jnp.float32),
    ],
)

_mid_tc = pl.pallas_call(
    _mid_body,
    grid=(N // R,),
    in_specs=[
        pl.BlockSpec((2, R, 128), lambda i: (0, i, 0)),
        pl.BlockSpec((R, 1), lambda i: (i, 0)),
        pl.BlockSpec((1, 256), lambda i: (0, 0)),
        pl.BlockSpec((1, 256), lambda i: (0, 0)),
        pl.BlockSpec((1, 256), lambda i: (0, 0)),
        pl.BlockSpec((256, 256), lambda i: (0, 0)),
    ],
    out_specs=pl.BlockSpec((2, R, 128), lambda i: (0, i, 0)),
    out_shape=jax.ShapeDtypeStruct((2, N, 128), jnp.float32),
)

_post_tc = pl.pallas_call(
    _post_body,
    grid=(N // R,),
    in_specs=[
        pl.BlockSpec((2, R, 128), lambda i: (0, i, 0)),
        pl.BlockSpec((R, 1), lambda i: (i, 0)),
        pl.BlockSpec((1, 256), lambda i: (0, 0)),
        pl.BlockSpec((1, 256), lambda i: (0, 0)),
        pl.BlockSpec((1, 256), lambda i: (0, 0)),
        pl.BlockSpec((R, 1), lambda i: (i, 0)),
        pl.BlockSpec((64, 1), lambda i: (0, 0)),
        pl.BlockSpec((256, 128), lambda i: (0, 0)),
        pl.BlockSpec((1, 128), lambda i: (0, 0)),
        pl.BlockSpec((1, 128), lambda i: (0, 0)),
    ],
    out_specs=pl.BlockSpec((64, 128), lambda i: (0, 0)),
    out_shape=jax.ShapeDtypeStruct((64, 128), jnp.float32),
    scratch_shapes=[
        pltpu.VMEM((64, 256), jnp.float32),
        pltpu.VMEM((64, 1), jnp.float32),
    ],
)


def kernel(x, edge_index, edge_attr, batch, logmw,
           W0, b0, W1, b1, W2, b2, gamma, beta, fcW, fcb):
    src = edge_index[0].astype(jnp.int32)
    dst = edge_index[1].astype(jnp.int32)
    ew = edge_attr.astype(jnp.float32)
    pad = E_PAD - src.shape[0]
    src_p = jnp.pad(src, (0, pad))
    dst_p = jnp.pad(dst, (0, pad))
    ew_p = jnp.pad(ew, (0, pad))
    ew16 = jnp.broadcast_to(ew_p[:, None], (E_PAD, 16))
    ones16 = jnp.ones((N, 16), jnp.float32)

    degp = _deg_kernel(ones16, dst_p, ew16).reshape(2, N, 16)
    g2, dis = _pre_tc(x, degp, W0)

    acc = _conv_kernel(g2.reshape(2 * N, 128), src_p, dst_p, ew16)
    g2 = _mid_tc(acc.reshape(2, N, 128), dis, b0.reshape(1, 256),
                 gamma.reshape(1, 256), beta.reshape(1, 256), W1)
    acc = _conv_kernel(g2.reshape(2 * N, 128), src_p, dst_p, ew16)
    g2 = _mid_tc(acc.reshape(2, N, 128), dis, b1.reshape(1, 256),
                 gamma.reshape(1, 256), beta.reshape(1, 256), W2)
    acc = _conv_kernel(g2.reshape(2 * N, 128), src_p, dst_p, ew16)

    return _post_tc(acc.reshape(2, N, 128), dis, b2.reshape(1, 256),
                    gamma.reshape(1, 256), beta.reshape(1, 256),
                    batch.astype(jnp.int32).reshape(N, 1), logmw,
                    fcW[:256], fcW[256:].reshape(1, 128),
                    fcb.reshape(1, 128))


# speedup vs baseline: 4.7292x; 1.0943x over previous
import functools

import jax
import jax.numpy as jnp
from jax import lax
from jax.experimental import pallas as pl
from jax.experimental.pallas import tpu as pltpu
from jax.experimental.pallas import tpu_sc as plsc

N = 10000
E_RAW = 160000
E_PAD = 163840
BLK = 64
RPT = 624
TAIL = N - 16 * RPT
EPT = E_PAD // 16
EPW = E_PAD // 32
R = 1000

_mesh = plsc.VectorSubcoreMesh(core_axis_name="c", subcore_axis_name="s")


@functools.partial(
    pl.kernel,
    mesh=_mesh,
    out_type=jax.ShapeDtypeStruct((2 * N, 16), jnp.float32),
    scratch_types=[
        pltpu.VMEM((BLK,), jnp.int32),
        pltpu.VMEM((BLK, 16), jnp.float32),
        pltpu.VMEM_SHARED((N, 16), jnp.float32),
        pltpu.SemaphoreType.DMA,
    ],
)
def _deg_kernel(ones_hbm, dst_hbm, ew16_hbm, out_hbm, didx, vals, acc, sem):
    c = lax.axis_index("c")
    s = lax.axis_index("s")
    w = s * 2 + c
    pltpu.sync_copy(ones_hbm.at[pl.ds(s * RPT, RPT)], acc.at[pl.ds(s * RPT, RPT)])

    @pl.when(s == 15)
    def _():
        pltpu.sync_copy(ones_hbm.at[pl.ds(16 * RPT, TAIL)],
                        acc.at[pl.ds(16 * RPT, TAIL)])

    plsc.subcore_barrier()

    def blk(j, carry):
        base = w * EPW + j * BLK
        pltpu.sync_copy(dst_hbm.at[pl.ds(base, BLK)], didx)
        pltpu.sync_copy(ew16_hbm.at[pl.ds(base, BLK)], vals)
        pltpu.sync_copy(vals, acc.at[didx], add=True)
        return carry

    lax.fori_loop(0, EPW // BLK, blk, 0)
    plsc.subcore_barrier()
    pltpu.sync_copy(
        acc.at[pl.ds(s * RPT, RPT)], out_hbm.at[pl.ds(c * N + s * RPT, RPT)]
    )

    @pl.when(s == 15)
    def _():
        pltpu.sync_copy(acc.at[pl.ds(16 * RPT, TAIL)],
                        out_hbm.at[pl.ds(c * N + 16 * RPT, TAIL)])


NBUF = 2


@functools.partial(
    pl.kernel,
    mesh=_mesh,
    out_type=jax.ShapeDtypeStruct((2 * N, 128), jnp.float32),
    scratch_types=(
        [pltpu.VMEM((BLK,), jnp.int32)] * NBUF
        + [pltpu.VMEM((BLK,), jnp.int32)] * NBUF
        + [pltpu.VMEM((BLK, 16), jnp.float32)] * NBUF
        + [pltpu.VMEM((BLK, 128), jnp.float32)] * NBUF
        + [pltpu.VMEM_SHARED((N, 128), jnp.float32)]
        + [pltpu.SemaphoreType.DMA] * (3 * NBUF)
    ),
)
def _conv_kernel(gflat_hbm, src_hbm, dst_hbm, ew16_hbm, out_hbm, *scr):
    sidx = scr[0:NBUF]
    didx = scr[NBUF:2 * NBUF]
    ewm = scr[2 * NBUF:3 * NBUF]
    rows = scr[3 * NBUF:4 * NBUF]
    acc = scr[4 * NBUF]
    sem_ld = scr[4 * NBUF + 1:4 * NBUF + 1 + NBUF]
    sem_g = scr[4 * NBUF + 1 + NBUF:4 * NBUF + 1 + 2 * NBUF]
    sem_sc = scr[4 * NBUF + 1 + 2 * NBUF:]

    c = lax.axis_index("c")
    s = lax.axis_index("s")
    off = c * N
    pltpu.sync_copy(
        gflat_hbm.at[pl.ds(off + s * RPT, RPT)], acc.at[pl.ds(s * RPT, RPT)]
    )

    @pl.when(s == 15)
    def _():
        pltpu.sync_copy(gflat_hbm.at[pl.ds(off + 16 * RPT, TAIL)],
                        acc.at[pl.ds(16 * RPT, TAIL)])

    plsc.subcore_barrier()

    def superblk(j, carry):
        lh = []
        for b in range(NBUF):
            @pl.when(j > 0)
            def _():
                pltpu.make_async_copy(
                    gflat_hbm.at[pl.ds(0, BLK)], rows[b], sem_sc[b]).wait()

            base = s * EPT + (j * NBUF + b) * BLK
            lh.append((
                pltpu.async_copy(src_hbm.at[pl.ds(base, BLK)], sidx[b], sem_ld[b]),
                pltpu.async_copy(dst_hbm.at[pl.ds(base, BLK)], didx[b], sem_ld[b]),
                pltpu.async_copy(ew16_hbm.at[pl.ds(base, BLK)], ewm[b], sem_ld[b]),
            ))
        gh = []
        for b in range(NBUF):
            for h in lh[b]:
                h.wait()
            for v in range(BLK // 16):
                sidx[b][pl.ds(v * 16, 16)] = sidx[b][pl.ds(v * 16, 16)] + off
            gh.append(pltpu.async_copy(gflat_hbm.at[sidx[b]], rows[b], sem_g[b]))
        for b in range(NBUF):
            gh[b].wait()
            rows_b, ewm_b = rows[b], ewm[b]

            def scale(jj, cc):
                wv = ewm_b[jj]
                for v in range(8):
                    rows_b[jj, pl.ds(v * 16, 16)] = (
                        rows_b[jj, pl.ds(v * 16, 16)] * wv)
                return cc

            lax.fori_loop(0, BLK, scale, 0, unroll=4)
            pltpu.async_copy(rows[b], acc.at[didx[b]], sem_sc[b], add=True)
        return carry

    lax.fori_loop(0, EPT // (BLK * NBUF), superblk, 0)
    for b in range(NBUF):
        pltpu.make_async_copy(
            gflat_hbm.at[pl.ds(0, BLK)], rows[b], sem_sc[b]).wait()
    plsc.subcore_barrier()
    pltpu.sync_copy(
        acc.at[pl.ds(s * RPT, RPT)], out_hbm.at[pl.ds(off + s * RPT, RPT)]
    )

    @pl.when(s == 15)
    def _():
        pltpu.sync_copy(acc.at[pl.ds(16 * RPT, TAIL)],
                        out_hbm.at[pl.ds(off + 16 * RPT, TAIL)])


def _pre_body(x_ref, degp_ref, w_ref, g2_ref, dis_ref):
    deg = degp_ref[0, :, 0:1] + degp_ref[1, :, 0:1] - 1.0
    dis = lax.rsqrt(deg)
    h = jnp.dot(x_ref[...], w_ref[...], preferred_element_type=jnp.float32)
    g = h * dis
    g2_ref[0] = g[:, :128]
    g2_ref[1] = g[:, 128:]
    dis_ref[...] = dis


def _norm_relu(acc_ref, dis, b_ref, gamma_ref, beta_ref):
    y = jnp.concatenate([acc_ref[0], acc_ref[1]], axis=1) * dis + b_ref[...]
    mu = jnp.mean(y, axis=-1, keepdims=True)
    var = jnp.mean((y - mu) * (y - mu), axis=-1, keepdims=True)
    yn = (y - mu) * lax.rsqrt(var + 1e-5) * gamma_ref[...] + beta_ref[...]
    return jnp.maximum(yn, 0.0)


def _mid_body(acc_ref, dis_ref, b_ref, gamma_ref, beta_ref, w_ref, g2_ref):
    dis = dis_ref[...]
    yr = _norm_relu(acc_ref, dis, b_ref, gamma_ref, beta_ref)
    g = jnp.dot(yr, w_ref[...], preferred_element_type=jnp.float32) * dis
    g2_ref[0] = g[:, :128]
    g2_ref[1] = g[:, 128:]


def _post_body(acc_ref, dis_ref, b_ref, gamma_ref, beta_ref, batch_ref,
               logmw_ref, fw1_ref, fw2_ref, fb_ref, out_ref, sums_ref, cnt_ref):
    i = pl.program_id(0)
    yr = _norm_relu(acc_ref, dis_ref[...], b_ref, gamma_ref, beta_ref)
    seg = batch_ref[...]
    gid = lax.broadcasted_iota(jnp.int32, (R, 64), 1)
    mask = (gid == seg).astype(jnp.float32)
    dn = (((0,), (0,)), ((), ()))
    psum = lax.dot_general(mask, yr, dn, preferred_element_type=jnp.float32)
    pcnt = lax.dot_general(mask, jnp.ones((R, 1), jnp.float32), dn,
                           preferred_element_type=jnp.float32)

    @pl.when(i == 0)
    def _():
        sums_ref[...] = psum
        cnt_ref[...] = pcnt

    @pl.when(i != 0)
    def _():
        sums_ref[...] = sums_ref[...] + psum
        cnt_ref[...] = cnt_ref[...] + pcnt

    @pl.when(i == (N // R) - 1)
    def _():
        pooled = sums_ref[...] / jnp.maximum(cnt_ref[...], 1.0)
        hv = (jnp.dot(pooled, fw1_ref[...], preferred_element_type=jnp.float32)
              + logmw_ref[...] * fw2_ref[...] + fb_ref[...])
        out_ref[...] = jnp.tanh(hv)


_pre_tc = pl.pallas_call(
    _pre_body,
    grid=(N // R,),
    in_specs=[
        pl.BlockSpec((R, 256), lambda i: (i, 0)),
        pl.BlockSpec((2, R, 16), lambda i: (0, i, 0)),
        pl.BlockSpec((256, 256), lambda i: (0, 0)),
    ],
    out_specs=[
        pl.BlockSpec((2, R, 128), lambda i: (0, i, 0)),
        pl.BlockSpec((R, 1), lambda i: (i, 0)),
    ],
    out_shape=[
        jax.ShapeDtypeStruct((2, N, 128), jnp.float32),
        jax.ShapeDtypeStruct((N, 1), jnp.float32),
    ],
)

_mid_tc = pl.pallas_call(
    _mid_body,
    grid=(N // R,),
    in_specs=[
        pl.BlockSpec((2, R, 128), lambda i: (0, i, 0)),
        pl.BlockSpec((R, 1), lambda i: (i, 0)),
        pl.BlockSpec((1, 256), lambda i: (0, 0)),
        pl.BlockSpec((1, 256), lambda i: (0, 0)),
        pl.BlockSpec((1, 256), lambda i: (0, 0)),
        pl.BlockSpec((256, 256), lambda i: (0, 0)),
    ],
    out_specs=pl.BlockSpec((2, R, 128), lambda i: (0, i, 0)),
    out_shape=jax.ShapeDtypeStruct((2, N, 128), jnp.float32),
)

_post_tc = pl.pallas_call(
    _post_body,
    grid=(N // R,),
    in_specs=[
        pl.BlockSpec((2, R, 128), lambda i: (0, i, 0)),
        pl.BlockSpec((R, 1), lambda i: (i, 0)),
        pl.BlockSpec((1, 256), lambda i: (0, 0)),
        pl.BlockSpec((1, 256), lambda i: (0, 0)),
        pl.BlockSpec((1, 256), lambda i: (0, 0)),
        pl.BlockSpec((R, 1), lambda i: (i, 0)),
        pl.BlockSpec((64, 1), lambda i: (0, 0)),
        pl.BlockSpec((256, 128), lambda i: (0, 0)),
        pl.BlockSpec((1, 128), lambda i: (0, 0)),
        pl.BlockSpec((1, 128), lambda i: (0, 0)),
    ],
    out_specs=pl.BlockSpec((64, 128), lambda i: (0, 0)),
    out_shape=jax.ShapeDtypeStruct((64, 128), jnp.float32),
    scratch_shapes=[
        pltpu.VMEM((64, 256), jnp.float32),
        pltpu.VMEM((64, 1), jnp.float32),
    ],
)


def kernel(x, edge_index, edge_attr, batch, logmw,
           W0, b0, W1, b1, W2, b2, gamma, beta, fcW, fcb):
    src = edge_index[0].astype(jnp.int32)
    dst = edge_index[1].astype(jnp.int32)
    ew = edge_attr.astype(jnp.float32)
    pad = E_PAD - src.shape[0]
    src_p = jnp.pad(src, (0, pad))
    dst_p = jnp.pad(dst, (0, pad))
    ew_p = jnp.pad(ew, (0, pad))
    ew16 = jnp.broadcast_to(ew_p[:, None], (E_PAD, 16))
    ones16 = jnp.ones((N, 16), jnp.float32)

    degp = _deg_kernel(ones16, dst_p, ew16).reshape(2, N, 16)
    g2, dis = _pre_tc(x, degp, W0)

    acc = _conv_kernel(g2.reshape(2 * N, 128), src_p, dst_p, ew16)
    g2 = _mid_tc(acc.reshape(2, N, 128), dis, b0.reshape(1, 256),
                 gamma.reshape(1, 256), beta.reshape(1, 256), W1)
    acc = _conv_kernel(g2.reshape(2 * N, 128), src_p, dst_p, ew16)
    g2 = _mid_tc(acc.reshape(2, N, 128), dis, b1.reshape(1, 256),
                 gamma.reshape(1, 256), beta.reshape(1, 256), W2)
    acc = _conv_kernel(g2.reshape(2 * N, 128), src_p, dst_p, ew16)

    return _post_tc(acc.reshape(2, N, 128), dis, b2.reshape(1, 256),
                    gamma.reshape(1, 256), beta.reshape(1, 256),
                    batch.astype(jnp.int32).reshape(N, 1), logmw,
                    fcW[:256], fcW[256:].reshape(1, 128),
                    fcb.reshape(1, 128))


# speedup vs baseline: 6.8540x; 1.4493x over previous
import functools

import jax
import jax.numpy as jnp
from jax import lax
from jax.experimental import pallas as pl
from jax.experimental.pallas import tpu as pltpu
from jax.experimental.pallas import tpu_sc as plsc

N = 10000
E_RAW = 160000
DBLK = 128
E_PADD = 163840
EPW = E_PADD // 32
BLK = 80
E_PADC = 161280
EPT = E_PADC // 16
NBT = EPT // BLK
RPT = 624
TAIL = N - 16 * RPT
R = 1000

_mesh = plsc.VectorSubcoreMesh(core_axis_name="c", subcore_axis_name="s")


@functools.partial(
    pl.kernel,
    mesh=_mesh,
    out_type=jax.ShapeDtypeStruct((2 * N, 16), jnp.float32),
    scratch_types=[
        pltpu.VMEM((DBLK,), jnp.int32),
        pltpu.VMEM((DBLK, 16), jnp.float32),
        pltpu.VMEM_SHARED((N, 16), jnp.float32),
        pltpu.SemaphoreType.DMA,
    ],
)
def _deg_kernel(ones_hbm, dst_hbm, ew16_hbm, out_hbm, didx, vals, acc, sem):
    c = lax.axis_index("c")
    s = lax.axis_index("s")
    w = s * 2 + c
    pltpu.sync_copy(ones_hbm.at[pl.ds(s * RPT, RPT)], acc.at[pl.ds(s * RPT, RPT)])

    @pl.when(s == 15)
    def _():
        pltpu.sync_copy(ones_hbm.at[pl.ds(16 * RPT, TAIL)],
                        acc.at[pl.ds(16 * RPT, TAIL)])

    plsc.subcore_barrier()

    def blk(j, carry):
        base = w * EPW + j * DBLK
        pltpu.sync_copy(dst_hbm.at[pl.ds(base, DBLK)], didx)
        pltpu.sync_copy(ew16_hbm.at[pl.ds(base, DBLK)], vals)
        pltpu.sync_copy(vals, acc.at[didx], add=True)
        return carry

    lax.fori_loop(0, EPW // DBLK, blk, 0)
    plsc.subcore_barrier()
    pltpu.sync_copy(
        acc.at[pl.ds(s * RPT, RPT)], out_hbm.at[pl.ds(c * N + s * RPT, RPT)]
    )

    @pl.when(s == 15)
    def _():
        pltpu.sync_copy(acc.at[pl.ds(16 * RPT, TAIL)],
                        out_hbm.at[pl.ds(c * N + 16 * RPT, TAIL)])


NBUF = 2


@functools.partial(
    pl.kernel,
    mesh=_mesh,
    out_type=jax.ShapeDtypeStruct((2 * N, 128), jnp.float32),
    scratch_types=(
        [pltpu.VMEM((4, BLK), jnp.int32)] * NBUF
        + [pltpu.VMEM((BLK, 16), jnp.float32)] * NBUF
        + [pltpu.VMEM((BLK, 128), jnp.float32)] * NBUF
        + [pltpu.VMEM_SHARED((N, 128), jnp.float32)]
        + [pltpu.SemaphoreType.DMA] * (4 * NBUF)
    ),
)
def _conv_kernel(gflat_hbm, packed_hbm, ewt_hbm, out_hbm, *scr):
    cbuf = scr[0:NBUF]
    ewt = scr[NBUF:2 * NBUF]
    rows = scr[2 * NBUF:3 * NBUF]
    acc = scr[3 * NBUF]
    sem_ld = scr[3 * NBUF + 1:3 * NBUF + 1 + NBUF]
    sem_ew = scr[3 * NBUF + 1 + NBUF:3 * NBUF + 1 + 2 * NBUF]
    sem_g = scr[3 * NBUF + 1 + 2 * NBUF:3 * NBUF + 1 + 3 * NBUF]
    sem_sc = scr[3 * NBUF + 1 + 3 * NBUF:]

    c = lax.axis_index("c")
    s = lax.axis_index("s")
    off = c * N
    pltpu.sync_copy(
        gflat_hbm.at[pl.ds(off + s * RPT, RPT)], acc.at[pl.ds(s * RPT, RPT)]
    )

    @pl.when(s == 15)
    def _():
        pltpu.sync_copy(gflat_hbm.at[pl.ds(off + 16 * RPT, TAIL)],
                        acc.at[pl.ds(16 * RPT, TAIL)])

    plsc.subcore_barrier()

    def superblk(j, carry):
        lh = []
        for b in range(NBUF):
            @pl.when(j > 0)
            def _():
                pltpu.make_async_copy(
                    gflat_hbm.at[pl.ds(0, BLK)], rows[b], sem_sc[b]).wait()

            blkid = s * NBT + j * NBUF + b
            lh.append(pltpu.async_copy(packed_hbm.at[blkid], cbuf[b],
                                       sem_ld[b]))
            pltpu.async_copy(ewt_hbm.at[blkid], ewt[b], sem_ew[b])
        gh = []
        for b in range(NBUF):
            lh[b].wait()
            for v in range(BLK // 16):
                cbuf[b][0, pl.ds(v * 16, 16)] = (
                    cbuf[b][0, pl.ds(v * 16, 16)] + off)
            gh.append(pltpu.async_copy(gflat_hbm.at[cbuf[b].at[0]], rows[b],
                                       sem_g[b]))
        for b in range(NBUF):
            gh[b].wait()
            pltpu.make_async_copy(ewt_hbm.at[0], ewt[b], sem_ew[b]).wait()
            rows_b, ewt_b = rows[b], ewt[b]
            for e in range(BLK):
                wv = ewt_b[e]
                for v in range(8):
                    rows_b[e, pl.ds(v * 16, 16)] = (
                        rows_b[e, pl.ds(v * 16, 16)] * wv)
            pltpu.async_copy(rows[b], acc.at[cbuf[b].at[1]], sem_sc[b],
                             add=True)
        return carry

    lax.fori_loop(0, NBT // NBUF, superblk, 0)
    for b in range(NBUF):
        pltpu.make_async_copy(
            gflat_hbm.at[pl.ds(0, BLK)], rows[b], sem_sc[b]).wait()
    plsc.subcore_barrier()
    pltpu.sync_copy(
        acc.at[pl.ds(s * RPT, RPT)], out_hbm.at[pl.ds(off + s * RPT, RPT)]
    )

    @pl.when(s == 15)
    def _():
        pltpu.sync_copy(acc.at[pl.ds(16 * RPT, TAIL)],
                        out_hbm.at[pl.ds(off + 16 * RPT, TAIL)])


def _pre_body(x_ref, degp_ref, w_ref, g2_ref, dis_ref):
    deg = degp_ref[0, :, 0:1] + degp_ref[1, :, 0:1] - 1.0
    dis = lax.rsqrt(deg)
    h = jnp.dot(x_ref[...], w_ref[...], preferred_element_type=jnp.float32)
    g = h * dis
    g2_ref[0] = g[:, :128]
    g2_ref[1] = g[:, 128:]
    dis_ref[...] = dis


def _norm_relu(acc_ref, dis, b_ref, gamma_ref, beta_ref):
    y = jnp.concatenate([acc_ref[0], acc_ref[1]], axis=1) * dis + b_ref[...]
    mu = jnp.mean(y, axis=-1, keepdims=True)
    var = jnp.mean((y - mu) * (y - mu), axis=-1, keepdims=True)
    yn = (y - mu) * lax.rsqrt(var + 1e-5) * gamma_ref[...] + beta_ref[...]
    return jnp.maximum(yn, 0.0)


def _mid_body(acc_ref, dis_ref, b_ref, gamma_ref, beta_ref, w_ref, g2_ref):
    dis = dis_ref[...]
    yr = _norm_relu(acc_ref, dis, b_ref, gamma_ref, beta_ref)
    g = jnp.dot(yr, w_ref[...], preferred_element_type=jnp.float32) * dis
    g2_ref[0] = g[:, :128]
    g2_ref[1] = g[:, 128:]


def _post_body(acc_ref, dis_ref, b_ref, gamma_ref, beta_ref, batch_ref,
               logmw_ref, fw1_ref, fw2_ref, fb_ref, out_ref, sums_ref, cnt_ref):
    i = pl.program_id(0)
    yr = _norm_relu(acc_ref, dis_ref[...], b_ref, gamma_ref, beta_ref)
    seg = batch_ref[...]
    gid = lax.broadcasted_iota(jnp.int32, (R, 64), 1)
    mask = (gid == seg).astype(jnp.float32)
    dn = (((0,), (0,)), ((), ()))
    psum = lax.dot_general(mask, yr, dn, preferred_element_type=jnp.float32)
    pcnt = lax.dot_general(mask, jnp.ones((R, 1), jnp.float32), dn,
                           preferred_element_type=jnp.float32)

    @pl.when(i == 0)
    def _():
        sums_ref[...] = psum
        cnt_ref[...] = pcnt

    @pl.when(i != 0)
    def _():
        sums_ref[...] = sums_ref[...] + psum
        cnt_ref[...] = cnt_ref[...] + pcnt

    @pl.when(i == (N // R) - 1)
    def _():
        pooled = sums_ref[...] / jnp.maximum(cnt_ref[...], 1.0)
        hv = (jnp.dot(pooled, fw1_ref[...], preferred_element_type=jnp.float32)
              + logmw_ref[...] * fw2_ref[...] + fb_ref[...])
        out_ref[...] = jnp.tanh(hv)


_pre_tc = pl.pallas_call(
    _pre_body,
    grid=(N // R,),
    in_specs=[
        pl.BlockSpec((R, 256), lambda i: (i, 0)),
        pl.BlockSpec((2, R, 16), lambda i: (0, i, 0)),
        pl.BlockSpec((256, 256), lambda i: (0, 0)),
    ],
    out_specs=[
        pl.BlockSpec((2, R, 128), lambda i: (0, i, 0)),
        pl.BlockSpec((R, 1), lambda i: (i, 0)),
    ],
    out_shape=[
        jax.ShapeDtypeStruct((2, N, 128), jnp.float32),
        jax.ShapeDtypeStruct((N, 1), jnp.float32),
    ],
)

_mid_tc = pl.pallas_call(
    _mid_body,
    grid=(N // R,),
    in_specs=[
        pl.BlockSpec((2, R, 128), lambda i: (0, i, 0)),
        pl.BlockSpec((R, 1), lambda i: (i, 0)),
        pl.BlockSpec((1, 256), lambda i: (0, 0)),
        pl.BlockSpec((1, 256), lambda i: (0, 0)),
        pl.BlockSpec((1, 256), lambda i: (0, 0)),
        pl.BlockSpec((256, 256), lambda i: (0, 0)),
    ],
    out_specs=pl.BlockSpec((2, R, 128), lambda i: (0, i, 0)),
    out_shape=jax.ShapeDtypeStruct((2, N, 128), jnp.float32),
)

_post_tc = pl.pallas_call(
    _post_body,
    grid=(N // R,),
    in_specs=[
        pl.BlockSpec((2, R, 128), lambda i: (0, i, 0)),
        pl.BlockSpec((R, 1), lambda i: (i, 0)),
        pl.BlockSpec((1, 256), lambda i: (0, 0)),
        pl.BlockSpec((1, 256), lambda i: (0, 0)),
        pl.BlockSpec((1, 256), lambda i: (0, 0)),
        pl.BlockSpec((R, 1), lambda i: (i, 0)),
        pl.BlockSpec((64, 1), lambda i: (0, 0)),
        pl.BlockSpec((256, 128), lambda i: (0, 0)),
        pl.BlockSpec((1, 128), lambda i: (0, 0)),
        pl.BlockSpec((1, 128), lambda i: (0, 0)),
    ],
    out_specs=pl.BlockSpec((64, 128), lambda i: (0, 0)),
    out_shape=jax.ShapeDtypeStruct((64, 128), jnp.float32),
    scratch_shapes=[
        pltpu.VMEM((64, 256), jnp.float32),
        pltpu.VMEM((64, 1), jnp.float32),
    ],
)


def kernel(x, edge_index, edge_attr, batch, logmw,
           W0, b0, W1, b1, W2, b2, gamma, beta, fcW, fcb):
    src = edge_index[0].astype(jnp.int32)
    dst = edge_index[1].astype(jnp.int32)
    ew = edge_attr.astype(jnp.float32)
    dst_d = jnp.pad(dst, (0, E_PADD - E_RAW))
    ew16 = jnp.broadcast_to(
        jnp.pad(ew, (0, E_PADD - E_RAW))[:, None], (E_PADD, 16))
    ones16 = jnp.ones((N, 16), jnp.float32)
    padc = E_PADC - E_RAW
    packed = jnp.stack([
        jnp.pad(src, (0, padc)),
        jnp.pad(dst, (0, padc)),
        jnp.zeros((E_PADC,), jnp.int32),
        jnp.zeros((E_PADC,), jnp.int32),
    ]).reshape(4, E_PADC // BLK, BLK).transpose(1, 0, 2)
    ewt = jnp.broadcast_to(
        jnp.pad(ew, (0, padc)).reshape(E_PADC // BLK, BLK, 1),
        (E_PADC // BLK, BLK, 16))

    degp = _deg_kernel(ones16, dst_d, ew16).reshape(2, N, 16)
    g2, dis = _pre_tc(x, degp, W0)

    acc = _conv_kernel(g2.reshape(2 * N, 128), packed, ewt)
    g2 = _mid_tc(acc.reshape(2, N, 128), dis, b0.reshape(1, 256),
                 gamma.reshape(1, 256), beta.reshape(1, 256), W1)
    acc = _conv_kernel(g2.reshape(2 * N, 128), packed, ewt)
    g2 = _mid_tc(acc.reshape(2, N, 128), dis, b1.reshape(1, 256),
                 gamma.reshape(1, 256), beta.reshape(1, 256), W2)
    acc = _conv_kernel(g2.reshape(2 * N, 128), packed, ewt)

    return _post_tc(acc.reshape(2, N, 128), dis, b2.reshape(1, 256),
                    gamma.reshape(1, 256), beta.reshape(1, 256),
                    batch.astype(jnp.int32).reshape(N, 1), logmw,
                    fcW[:256], fcW[256:].reshape(1, 128),
                    fcb.reshape(1, 128))


# speedup vs baseline: 7.2475x; 1.0574x over previous
import functools

import jax
import jax.numpy as jnp
from jax import lax
from jax.experimental import pallas as pl
from jax.experimental.pallas import tpu as pltpu
from jax.experimental.pallas import tpu_sc as plsc

N = 10000
E_RAW = 160000
DBLK = 128
E_PADD = 163840
EPW = E_PADD // 32
BLK = 80
E_PADC = 161280
EPT = E_PADC // 16
NBT = EPT // BLK
RPT = 624
TAIL = N - 16 * RPT
R = 1000

_mesh = plsc.VectorSubcoreMesh(core_axis_name="c", subcore_axis_name="s")


@functools.partial(
    pl.kernel,
    mesh=_mesh,
    out_type=jax.ShapeDtypeStruct((2 * N, 16), jnp.float32),
    scratch_types=[
        pltpu.VMEM((DBLK,), jnp.int32),
        pltpu.VMEM((DBLK, 16), jnp.float32),
        pltpu.VMEM_SHARED((N, 16), jnp.float32),
        pltpu.SemaphoreType.DMA,
    ],
)
def _deg_kernel(ones_hbm, dst_hbm, ew16_hbm, out_hbm, didx, vals, acc, sem):
    c = lax.axis_index("c")
    s = lax.axis_index("s")
    w = s * 2 + c
    pltpu.sync_copy(ones_hbm.at[pl.ds(s * RPT, RPT)], acc.at[pl.ds(s * RPT, RPT)])

    @pl.when(s == 15)
    def _():
        pltpu.sync_copy(ones_hbm.at[pl.ds(16 * RPT, TAIL)],
                        acc.at[pl.ds(16 * RPT, TAIL)])

    plsc.subcore_barrier()

    def blk(j, carry):
        base = w * EPW + j * DBLK
        pltpu.sync_copy(dst_hbm.at[pl.ds(base, DBLK)], didx)
        pltpu.sync_copy(ew16_hbm.at[pl.ds(base, DBLK)], vals)
        pltpu.sync_copy(vals, acc.at[didx], add=True)
        return carry

    lax.fori_loop(0, EPW // DBLK, blk, 0)
    plsc.subcore_barrier()
    pltpu.sync_copy(
        acc.at[pl.ds(s * RPT, RPT)], out_hbm.at[pl.ds(c * N + s * RPT, RPT)]
    )

    @pl.when(s == 15)
    def _():
        pltpu.sync_copy(acc.at[pl.ds(16 * RPT, TAIL)],
                        out_hbm.at[pl.ds(c * N + 16 * RPT, TAIL)])


NBUF = 2


@functools.partial(
    pl.kernel,
    mesh=_mesh,
    out_type=jax.ShapeDtypeStruct((2 * N, 128), jnp.float32),
    scratch_types=(
        [pltpu.VMEM((4, BLK), jnp.int32)] * NBUF
        + [pltpu.VMEM((BLK, 16), jnp.float32)] * NBUF
        + [pltpu.VMEM((BLK, 128), jnp.float32)] * NBUF
        + [pltpu.VMEM_SHARED((N, 128), jnp.float32)]
        + [pltpu.SemaphoreType.DMA] * (4 * NBUF)
    ),
)
def _conv_kernel(gflat_hbm, packed_hbm, ewt_hbm, out_hbm, *scr):
    cbuf = scr[0:NBUF]
    ewt = scr[NBUF:2 * NBUF]
    rows = scr[2 * NBUF:3 * NBUF]
    acc = scr[3 * NBUF]
    sem_ld = scr[3 * NBUF + 1:3 * NBUF + 1 + NBUF]
    sem_ew = scr[3 * NBUF + 1 + NBUF:3 * NBUF + 1 + 2 * NBUF]
    sem_g = scr[3 * NBUF + 1 + 2 * NBUF:3 * NBUF + 1 + 3 * NBUF]
    sem_sc = scr[3 * NBUF + 1 + 3 * NBUF:]

    c = lax.axis_index("c")
    s = lax.axis_index("s")
    off = c * N
    pltpu.sync_copy(
        gflat_hbm.at[pl.ds(off + s * RPT, RPT)], acc.at[pl.ds(s * RPT, RPT)]
    )

    @pl.when(s == 15)
    def _():
        pltpu.sync_copy(gflat_hbm.at[pl.ds(off + 16 * RPT, TAIL)],
                        acc.at[pl.ds(16 * RPT, TAIL)])

    plsc.subcore_barrier()

    def superblk(j, carry):
        lh = []
        for b in range(NBUF):
            @pl.when(j > 0)
            def _():
                pltpu.make_async_copy(
                    gflat_hbm.at[pl.ds(0, BLK)], rows[b], sem_sc[b]).wait()

            blkid = s * NBT + j * NBUF + b
            lh.append(pltpu.async_copy(packed_hbm.at[blkid], cbuf[b],
                                       sem_ld[b]))
            pltpu.async_copy(ewt_hbm.at[blkid], ewt[b], sem_ew[b])
        gh = []
        for b in range(NBUF):
            lh[b].wait()
            for v in range(BLK // 16):
                cbuf[b][0, pl.ds(v * 16, 16)] = (
                    cbuf[b][0, pl.ds(v * 16, 16)] + off)
            gh.append(pltpu.async_copy(gflat_hbm.at[cbuf[b].at[0]], rows[b],
                                       sem_g[b]))
        for b in range(NBUF):
            gh[b].wait()
            pltpu.make_async_copy(ewt_hbm.at[0], ewt[b], sem_ew[b]).wait()
            rows_b, ewt_b = rows[b], ewt[b]
            for q in range(BLK // 16):
                wvs = [ewt_b[q * 16 + t] for t in range(16)]
                for t in range(16):
                    e = q * 16 + t
                    for v in range(8):
                        rows_b[e, pl.ds(v * 16, 16)] = (
                            rows_b[e, pl.ds(v * 16, 16)] * wvs[t])
            pltpu.async_copy(rows[b], acc.at[cbuf[b].at[1]], sem_sc[b],
                             add=True)
        return carry

    lax.fori_loop(0, NBT // NBUF, superblk, 0)
    for b in range(NBUF):
        pltpu.make_async_copy(
            gflat_hbm.at[pl.ds(0, BLK)], rows[b], sem_sc[b]).wait()
    plsc.subcore_barrier()
    pltpu.sync_copy(
        acc.at[pl.ds(s * RPT, RPT)], out_hbm.at[pl.ds(off + s * RPT, RPT)]
    )

    @pl.when(s == 15)
    def _():
        pltpu.sync_copy(acc.at[pl.ds(16 * RPT, TAIL)],
                        out_hbm.at[pl.ds(off + 16 * RPT, TAIL)])


def _pre_body(x_ref, degp_ref, w_ref, g2_ref, dis_ref):
    deg = degp_ref[0, :, 0:1] + degp_ref[1, :, 0:1] - 1.0
    dis = lax.rsqrt(deg)
    h = jnp.dot(x_ref[...], w_ref[...], preferred_element_type=jnp.float32)
    g = h * dis
    g2_ref[0] = g[:, :128]
    g2_ref[1] = g[:, 128:]
    dis_ref[...] = dis


def _norm_relu(acc_ref, dis, b_ref, gamma_ref, beta_ref):
    y = jnp.concatenate([acc_ref[0], acc_ref[1]], axis=1) * dis + b_ref[...]
    mu = jnp.mean(y, axis=-1, keepdims=True)
    var = jnp.mean((y - mu) * (y - mu), axis=-1, keepdims=True)
    yn = (y - mu) * lax.rsqrt(var + 1e-5) * gamma_ref[...] + beta_ref[...]
    return jnp.maximum(yn, 0.0)


def _mid_body(acc_ref, dis_ref, b_ref, gamma_ref, beta_ref, w_ref, g2_ref):
    dis = dis_ref[...]
    yr = _norm_relu(acc_ref, dis, b_ref, gamma_ref, beta_ref)
    g = jnp.dot(yr, w_ref[...], preferred_element_type=jnp.float32) * dis
    g2_ref[0] = g[:, :128]
    g2_ref[1] = g[:, 128:]


def _post_body(acc_ref, dis_ref, b_ref, gamma_ref, beta_ref, batch_ref,
               logmw_ref, fw1_ref, fw2_ref, fb_ref, out_ref, sums_ref, cnt_ref):
    i = pl.program_id(0)
    yr = _norm_relu(acc_ref, dis_ref[...], b_ref, gamma_ref, beta_ref)
    seg = batch_ref[...]
    gid = lax.broadcasted_iota(jnp.int32, (R, 64), 1)
    mask = (gid == seg).astype(jnp.float32)
    dn = (((0,), (0,)), ((), ()))
    psum = lax.dot_general(mask, yr, dn, preferred_element_type=jnp.float32)
    pcnt = lax.dot_general(mask, jnp.ones((R, 1), jnp.float32), dn,
                           preferred_element_type=jnp.float32)

    @pl.when(i == 0)
    def _():
        sums_ref[...] = psum
        cnt_ref[...] = pcnt

    @pl.when(i != 0)
    def _():
        sums_ref[...] = sums_ref[...] + psum
        cnt_ref[...] = cnt_ref[...] + pcnt

    @pl.when(i == (N // R) - 1)
    def _():
        pooled = sums_ref[...] / jnp.maximum(cnt_ref[...], 1.0)
        hv = (jnp.dot(pooled, fw1_ref[...], preferred_element_type=jnp.float32)
              + logmw_ref[...] * fw2_ref[...] + fb_ref[...])
        out_ref[...] = jnp.tanh(hv)


_pre_tc = pl.pallas_call(
    _pre_body,
    grid=(N // R,),
    in_specs=[
        pl.BlockSpec((R, 256), lambda i: (i, 0)),
        pl.BlockSpec((2, R, 16), lambda i: (0, i, 0)),
        pl.BlockSpec((256, 256), lambda i: (0, 0)),
    ],
    out_specs=[
        pl.BlockSpec((2, R, 128), lambda i: (0, i, 0)),
        pl.BlockSpec((R, 1), lambda i: (i, 0)),
    ],
    out_shape=[
        jax.ShapeDtypeStruct((2, N, 128), jnp.float32),
        jax.ShapeDtypeStruct((N, 1), jnp.float32),
    ],
)

_mid_tc = pl.pallas_call(
    _mid_body,
    grid=(N // R,),
    in_specs=[
        pl.BlockSpec((2, R, 128), lambda i: (0, i, 0)),
        pl.BlockSpec((R, 1), lambda i: (i, 0)),
        pl.BlockSpec((1, 256), lambda i: (0, 0)),
        pl.BlockSpec((1, 256), lambda i: (0, 0)),
        pl.BlockSpec((1, 256), lambda i: (0, 0)),
        pl.BlockSpec((256, 256), lambda i: (0, 0)),
    ],
    out_specs=pl.BlockSpec((2, R, 128), lambda i: (0, i, 0)),
    out_shape=jax.ShapeDtypeStruct((2, N, 128), jnp.float32),
)

_post_tc = pl.pallas_call(
    _post_body,
    grid=(N // R,),
    in_specs=[
        pl.BlockSpec((2, R, 128), lambda i: (0, i, 0)),
        pl.BlockSpec((R, 1), lambda i: (i, 0)),
        pl.BlockSpec((1, 256), lambda i: (0, 0)),
        pl.BlockSpec((1, 256), lambda i: (0, 0)),
        pl.BlockSpec((1, 256), lambda i: (0, 0)),
        pl.BlockSpec((R, 1), lambda i: (i, 0)),
        pl.BlockSpec((64, 1), lambda i: (0, 0)),
        pl.BlockSpec((256, 128), lambda i: (0, 0)),
        pl.BlockSpec((1, 128), lambda i: (0, 0)),
        pl.BlockSpec((1, 128), lambda i: (0, 0)),
    ],
    out_specs=pl.BlockSpec((64, 128), lambda i: (0, 0)),
    out_shape=jax.ShapeDtypeStruct((64, 128), jnp.float32),
    scratch_shapes=[
        pltpu.VMEM((64, 256), jnp.float32),
        pltpu.VMEM((64, 1), jnp.float32),
    ],
)


def kernel(x, edge_index, edge_attr, batch, logmw,
           W0, b0, W1, b1, W2, b2, gamma, beta, fcW, fcb):
    src = edge_index[0].astype(jnp.int32)
    dst = edge_index[1].astype(jnp.int32)
    ew = edge_attr.astype(jnp.float32)
    dst_d = jnp.pad(dst, (0, E_PADD - E_RAW))
    ew16 = jnp.broadcast_to(
        jnp.pad(ew, (0, E_PADD - E_RAW))[:, None], (E_PADD, 16))
    ones16 = jnp.ones((N, 16), jnp.float32)
    padc = E_PADC - E_RAW
    packed = jnp.stack([
        jnp.pad(src, (0, padc)),
        jnp.pad(dst, (0, padc)),
        jnp.zeros((E_PADC,), jnp.int32),
        jnp.zeros((E_PADC,), jnp.int32),
    ]).reshape(4, E_PADC // BLK, BLK).transpose(1, 0, 2)
    ewt = jnp.broadcast_to(
        jnp.pad(ew, (0, padc)).reshape(E_PADC // BLK, BLK, 1),
        (E_PADC // BLK, BLK, 16))

    degp = _deg_kernel(ones16, dst_d, ew16).reshape(2, N, 16)
    g2, dis = _pre_tc(x, degp, W0)

    acc = _conv_kernel(g2.reshape(2 * N, 128), packed, ewt)
    g2 = _mid_tc(acc.reshape(2, N, 128), dis, b0.reshape(1, 256),
                 gamma.reshape(1, 256), beta.reshape(1, 256), W1)
    acc = _conv_kernel(g2.reshape(2 * N, 128), packed, ewt)
    g2 = _mid_tc(acc.reshape(2, N, 128), dis, b1.reshape(1, 256),
                 gamma.reshape(1, 256), beta.reshape(1, 256), W2)
    acc = _conv_kernel(g2.reshape(2 * N, 128), packed, ewt)

    return _post_tc(acc.reshape(2, N, 128), dis, b2.reshape(1, 256),
                    gamma.reshape(1, 256), beta.reshape(1, 256),
                    batch.astype(jnp.int32).reshape(N, 1), logmw,
                    fcW[:256], fcW[256:].reshape(1, 128),
                    fcb.reshape(1, 128))
